# baseline (speedup 1.0000x reference)
import jax
import jax.numpy as jnp
from jax import lax
from jax.experimental import pallas as pl
from jax.experimental.pallas import tpu as pltpu
from jax.experimental.pallas import tpu_sc as plsc

_BATCH = 16384
_HIST = 200
_EMBED = 16
_N = _BATCH * _HIST

_HT = _HIST // 8
_BT = _BATCH // 128
_TILES = _HT * _BT
_TPS = 2


def _sc_gather(table, idx_phys):
    vector_mesh = plsc.VectorSubcoreMesh(
        core_axis_name="core", subcore_axis_name="subcore"
    )

    @pl.kernel(
        out_type=jax.ShapeDtypeStruct((_N, _EMBED), jnp.float32),
        mesh=vector_mesh,
        scratch_types=[pltpu.VMEM((_TPS * 1024,), jnp.int32)],
        compiler_params=pltpu.CompilerParams(
            use_tc_tiling_on_sc=False, needs_layout_passes=False
        ),
    )
    def gather_kernel(x_hbm, i_hbm, o_hbm, idx_t):
        core_id = lax.axis_index("core")
        steps_per_core = _TILES // _TPS // 2
        step0 = core_id * steps_per_core
        iota16 = lax.iota(jnp.int32, 16)
        base_vec = (iota16 % 8) * 128 + iota16 // 8

        def body(i_vmem, o_vmem):
            def v_body(v, carry):
                k = v // 64
                ids = base_vec + 2 * (v % 64)
                vec = plsc.load_gather(i_vmem, [iota16 * 0 + k, ids])
                idx_t[pl.ds(v * 16, 16)] = vec
                return carry

            lax.fori_loop(0, _TPS * 64, v_body, 0)
            pltpu.sync_copy(x_hbm.at[idx_t], o_vmem)

        pltpu.emit_pipeline(
            body,
            grid=(steps_per_core,),
            in_specs=[
                pl.BlockSpec((_TPS, 1024), index_map=lambda i: (step0 + i, 0))
            ],
            out_specs=[
                pl.BlockSpec(
                    (_TPS * 1024, _EMBED),
                    index_map=lambda i: (step0 + i, 0),
                )
            ],
            core_axis_name="subcore",
            dimension_semantics=(pltpu.PARALLEL,),
        )(i_hbm, o_hbm)

    return gather_kernel(table, idx_phys)


def _tc_permute(rows_view):
    def body(in_ref, out_ref):
        x = in_ref[...]
        out_ref[...] = x.T.reshape(8, 2, 1, 8, 128)

    return pl.pallas_call(
        body,
        grid=(_TILES,),
        in_specs=[pl.BlockSpec((128, 128), lambda i: (i, 0))],
        out_specs=pl.BlockSpec(
            (8, 2, 1, 8, 128),
            lambda i: (i // _BT, 0, i % _BT, 0, 0),
        ),
        out_shape=jax.ShapeDtypeStruct((_HIST, 2, _BT, 8, 128), jnp.float32),
    )(rows_view)


def kernel(pitch_type, table):
    idx_phys = (
        pitch_type.T.reshape(_HT, 8, _BT, 128)
        .transpose(0, 2, 1, 3)
        .reshape(_TILES, 1024)
        .astype(jnp.int32)
    )

    rows = _sc_gather(table, idx_phys)
    rows_view = rows.reshape(409600, 128)
    out_phys = _tc_permute(rows_view)

    return out_phys.transpose(2, 4, 0, 1, 3).reshape(_BATCH, _HIST, _EMBED)


# speedup vs baseline: 2.6877x; 2.6877x over previous
import jax
import jax.numpy as jnp
from jax import lax
from jax.experimental import pallas as pl
from jax.experimental.pallas import tpu as pltpu
from jax.experimental.pallas import tpu_sc as plsc

_BATCH = 16384
_HIST = 200
_EMBED = 16
_N = _BATCH * _HIST

_HT = _HIST // 8
_BT = _BATCH // 128
_TILES = _HT * _BT
_TPS = 2


def _sc_gather(table, idx_phys):
    vector_mesh = plsc.VectorSubcoreMesh(
        core_axis_name="core", subcore_axis_name="subcore"
    )

    @pl.kernel(
        out_type=jax.ShapeDtypeStruct((_N, _EMBED), jnp.float32),
        mesh=vector_mesh,
        scratch_types=[pltpu.VMEM((_TPS * 1024,), jnp.int32)],
        compiler_params=pltpu.CompilerParams(
            use_tc_tiling_on_sc=False, needs_layout_passes=False
        ),
    )
    def gather_kernel(x_hbm, i_hbm, o_hbm, idx_t):
        core_id = lax.axis_index("core")
        steps_per_core = _TILES // _TPS // 2
        step0 = core_id * steps_per_core
        iota16 = lax.iota(jnp.int32, 16)
        base_vec = (iota16 % 8) * 128 + iota16 // 8

        def body(i_vmem, o_vmem):
            def v_body(v, carry):
                k = v // 64
                ids = base_vec + 2 * (v % 64)
                vec = plsc.load_gather(i_vmem, [iota16 * 0 + k, ids])
                idx_t[pl.ds(v * 16, 16)] = vec
                return carry

            lax.fori_loop(0, _TPS * 64, v_body, 0)
            pltpu.sync_copy(x_hbm.at[idx_t], o_vmem)

        pltpu.emit_pipeline(
            body,
            grid=(steps_per_core,),
            in_specs=[
                pl.BlockSpec((_TPS, 1024), index_map=lambda i: (step0 + i, 0))
            ],
            out_specs=[
                pl.BlockSpec(
                    (_TPS * 1024, _EMBED),
                    index_map=lambda i: (step0 + i, 0),
                )
            ],
            core_axis_name="subcore",
            dimension_semantics=(pltpu.PARALLEL,),
        )(i_hbm, o_hbm)

    return gather_kernel(table, idx_phys)


_TC_STEP = 16


def _tc_permute(rows_view):
    def body(in_ref, out_ref):
        x = in_ref[...]
        x3 = x.reshape(_TC_STEP, 128, 128)
        y3 = x3.transpose(0, 2, 1)
        y5 = y3.reshape(_TC_STEP, 8, 2, 8, 128)
        out_ref[...] = y5.transpose(1, 2, 0, 3, 4)

    return pl.pallas_call(
        body,
        grid=(_TILES // _TC_STEP,),
        in_specs=[pl.BlockSpec((_TC_STEP * 128, 128), lambda i: (i, 0))],
        out_specs=pl.BlockSpec(
            (8, 2, _TC_STEP, 8, 128),
            lambda i: (i // (_BT // _TC_STEP), 0, i % (_BT // _TC_STEP), 0, 0),
        ),
        out_shape=jax.ShapeDtypeStruct((_HIST, 2, _BT, 8, 128), jnp.float32),
    )(rows_view)


def kernel(pitch_type, table):
    idx_phys = (
        pitch_type.T.reshape(_HT, 8, _BT, 128)
        .transpose(0, 2, 1, 3)
        .reshape(_TILES, 1024)
        .astype(jnp.int32)
    )

    rows = _sc_gather(table, idx_phys)
    rows_view = rows.reshape(409600, 128)
    out_phys = _tc_permute(rows_view)

    return out_phys.transpose(2, 4, 0, 1, 3).reshape(_BATCH, _HIST, _EMBED)


# speedup vs baseline: 2.8717x; 1.0685x over previous
import jax
import jax.numpy as jnp
from jax import lax
from jax.experimental import pallas as pl
from jax.experimental.pallas import tpu as pltpu
from jax.experimental.pallas import tpu_sc as plsc

_BATCH = 16384
_HIST = 200
_EMBED = 16
_N = _BATCH * _HIST

_HT = _HIST // 8
_BT = _BATCH // 128
_TILES = _HT * _BT
_TPS = 2


def _sc_gather(table, idx_phys):
    vector_mesh = plsc.VectorSubcoreMesh(
        core_axis_name="core", subcore_axis_name="subcore"
    )

    @pl.kernel(
        out_type=jax.ShapeDtypeStruct((_N, _EMBED), jnp.float32),
        mesh=vector_mesh,
        scratch_types=[pltpu.VMEM((_TPS * 1024,), jnp.int32)],
        compiler_params=pltpu.CompilerParams(
            use_tc_tiling_on_sc=False, needs_layout_passes=False
        ),
    )
    def gather_kernel(x_hbm, i_hbm, o_hbm, idx_t):
        core_id = lax.axis_index("core")
        steps_per_core = _TILES // _TPS // 2
        step0 = core_id * steps_per_core
        iota16 = lax.iota(jnp.int32, 16)
        base_vec = (iota16 % 8) * 128 + iota16 // 8

        def body(i_vmem, o_vmem):
            def v_body(v, carry):
                k = v // 64
                ids = base_vec + 2 * (v % 64)
                vec = plsc.load_gather(i_vmem, [iota16 * 0 + k, ids])
                idx_t[pl.ds(v * 16, 16)] = vec
                return carry

            lax.fori_loop(0, _TPS * 64, v_body, 0)
            pltpu.sync_copy(x_hbm.at[idx_t], o_vmem)

        pltpu.emit_pipeline(
            body,
            grid=(steps_per_core,),
            in_specs=[
                pl.BlockSpec((_TPS, 1024), index_map=lambda i: (step0 + i, 0))
            ],
            out_specs=[
                pl.BlockSpec(
                    (_TPS * 1024, _EMBED),
                    index_map=lambda i: (step0 + i, 0),
                )
            ],
            core_axis_name="subcore",
            dimension_semantics=(pltpu.PARALLEL,),
        )(i_hbm, o_hbm)

    return gather_kernel(table, idx_phys)


_TC_STEP = 32


def _tc_permute(rows_view):
    def body(in_ref, out_ref):
        x = in_ref[...]
        x3 = x.reshape(_TC_STEP, 128, 128)
        y3 = x3.transpose(0, 2, 1)
        y5 = y3.reshape(_TC_STEP, 8, 2, 8, 128)
        out_ref[...] = y5.transpose(1, 2, 0, 3, 4)

    return pl.pallas_call(
        body,
        grid=(_TILES // _TC_STEP,),
        in_specs=[pl.BlockSpec((_TC_STEP * 128, 128), lambda i: (i, 0))],
        out_specs=pl.BlockSpec(
            (8, 2, _TC_STEP, 8, 128),
            lambda i: (i // (_BT // _TC_STEP), 0, i % (_BT // _TC_STEP), 0, 0),
        ),
        out_shape=jax.ShapeDtypeStruct((_HIST, 2, _BT, 8, 128), jnp.float32),
    )(rows_view)


def kernel(pitch_type, table):
    idx_phys = (
        pitch_type.T.reshape(_HT, 8, _BT, 128)
        .transpose(0, 2, 1, 3)
        .reshape(_TILES, 1024)
        .astype(jnp.int32)
    )

    rows = _sc_gather(table, idx_phys)
    rows_view = rows.reshape(409600, 128)
    out_phys = _tc_permute(rows_view)

    return out_phys.transpose(2, 4, 0, 1, 3).reshape(_BATCH, _HIST, _EMBED)


# speedup vs baseline: 2.9679x; 1.0335x over previous
import jax
import jax.numpy as jnp
from jax import lax
from jax.experimental import pallas as pl
from jax.experimental.pallas import tpu as pltpu
from jax.experimental.pallas import tpu_sc as plsc

_BATCH = 16384
_HIST = 200
_EMBED = 16
_N = _BATCH * _HIST

_HT = _HIST // 8
_BT = _BATCH // 128
_TILES = _HT * _BT
_TPS = 2


def _sc_gather(table, idx_phys):
    vector_mesh = plsc.VectorSubcoreMesh(
        core_axis_name="core", subcore_axis_name="subcore"
    )

    @pl.kernel(
        out_type=jax.ShapeDtypeStruct((_N, _EMBED), jnp.float32),
        mesh=vector_mesh,
        scratch_types=[pltpu.VMEM((_TPS * 1024,), jnp.int32)],
        compiler_params=pltpu.CompilerParams(
            use_tc_tiling_on_sc=False, needs_layout_passes=False
        ),
    )
    def gather_kernel(x_hbm, i_hbm, o_hbm, idx_t):
        core_id = lax.axis_index("core")
        steps_per_core = _TILES // _TPS // 2
        step0 = core_id * steps_per_core
        iota16 = lax.iota(jnp.int32, 16)
        base_vec = (iota16 % 8) * 128 + iota16 // 8

        def body(i_vmem, o_vmem):
            def v_body(v, carry):
                k = v // 64
                ids = base_vec + 2 * (v % 64)
                vec = plsc.load_gather(i_vmem, [iota16 * 0 + k, ids])
                idx_t[pl.ds(v * 16, 16)] = vec
                return carry

            lax.fori_loop(0, _TPS * 64, v_body, 0)
            pltpu.sync_copy(x_hbm.at[idx_t], o_vmem)

        pltpu.emit_pipeline(
            body,
            grid=(steps_per_core,),
            in_specs=[
                pl.BlockSpec((_TPS, 1024), index_map=lambda i: (step0 + i, 0))
            ],
            out_specs=[
                pl.BlockSpec(
                    (_TPS * 1024, _EMBED),
                    index_map=lambda i: (step0 + i, 0),
                )
            ],
            core_axis_name="subcore",
            dimension_semantics=(pltpu.PARALLEL,),
        )(i_hbm, o_hbm)

    return gather_kernel(table, idx_phys)


_TC_STEP = 64


def _tc_permute(rows_view):
    def body(in_ref, out_ref):
        x = in_ref[...]
        x3 = x.reshape(_TC_STEP, 128, 128)
        y3 = x3.transpose(0, 2, 1)
        y5 = y3.reshape(_TC_STEP, 8, 2, 8, 128)
        out_ref[...] = y5.transpose(1, 2, 0, 3, 4)

    return pl.pallas_call(
        body,
        grid=(_TILES // _TC_STEP,),
        in_specs=[pl.BlockSpec((_TC_STEP * 128, 128), lambda i: (i, 0))],
        out_specs=pl.BlockSpec(
            (8, 2, _TC_STEP, 8, 128),
            lambda i: (i // (_BT // _TC_STEP), 0, i % (_BT // _TC_STEP), 0, 0),
        ),
        out_shape=jax.ShapeDtypeStruct((_HIST, 2, _BT, 8, 128), jnp.float32),
    )(rows_view)


def kernel(pitch_type, table):
    idx_phys = (
        pitch_type.T.reshape(_HT, 8, _BT, 128)
        .transpose(0, 2, 1, 3)
        .reshape(_TILES, 1024)
        .astype(jnp.int32)
    )

    rows = _sc_gather(table, idx_phys)
    rows_view = rows.reshape(409600, 128)
    out_phys = _tc_permute(rows_view)

    return out_phys.transpose(2, 4, 0, 1, 3).reshape(_BATCH, _HIST, _EMBED)


# speedup vs baseline: 2.9797x; 1.0040x over previous
import jax
import jax.numpy as jnp
from jax import lax
from jax.experimental import pallas as pl
from jax.experimental.pallas import tpu as pltpu
from jax.experimental.pallas import tpu_sc as plsc

_BATCH = 16384
_HIST = 200
_EMBED = 16
_N = _BATCH * _HIST

_HT = _HIST // 8
_BT = _BATCH // 128
_TILES = _HT * _BT
_TPS = 2


def _sc_gather(table, idx_phys):
    vector_mesh = plsc.VectorSubcoreMesh(
        core_axis_name="core", subcore_axis_name="subcore"
    )

    @pl.kernel(
        out_type=jax.ShapeDtypeStruct((_N, _EMBED), jnp.float32),
        mesh=vector_mesh,
        scratch_types=[pltpu.VMEM((_TPS * 1024,), jnp.int32)],
        compiler_params=pltpu.CompilerParams(
            use_tc_tiling_on_sc=False, needs_layout_passes=False
        ),
    )
    def gather_kernel(x_hbm, i_hbm, o_hbm, idx_t):
        core_id = lax.axis_index("core")
        steps_per_core = _TILES // _TPS // 2
        step0 = core_id * steps_per_core
        iota16 = lax.iota(jnp.int32, 16)
        base_vec = (iota16 % 8) * 128 + iota16 // 8

        def body(i_vmem, o_vmem):
            def v_body(v, carry):
                k = v // 64
                ids = base_vec + 2 * (v % 64)
                vec = plsc.load_gather(i_vmem, [iota16 * 0 + k, ids])
                idx_t[pl.ds(v * 16, 16)] = vec
                return carry

            lax.fori_loop(0, _TPS * 64, v_body, 0)
            pltpu.sync_copy(x_hbm.at[idx_t], o_vmem)

        pltpu.emit_pipeline(
            body,
            grid=(steps_per_core,),
            in_specs=[
                pl.BlockSpec((_TPS, 1024), index_map=lambda i: (step0 + i, 0))
            ],
            out_specs=[
                pl.BlockSpec(
                    (_TPS * 1024, _EMBED),
                    index_map=lambda i: (step0 + i, 0),
                )
            ],
            core_axis_name="subcore",
            dimension_semantics=(pltpu.PARALLEL,),
        )(i_hbm, o_hbm)

    return gather_kernel(table, idx_phys)


_TC_STEP = 128


def _tc_permute(rows_view):
    def body(in_ref, out_ref):
        x = in_ref[...]
        x3 = x.reshape(_TC_STEP, 128, 128)
        y3 = x3.transpose(0, 2, 1)
        y5 = y3.reshape(_TC_STEP, 8, 2, 8, 128)
        out_ref[...] = y5.transpose(1, 2, 0, 3, 4)

    return pl.pallas_call(
        body,
        grid=(_TILES // _TC_STEP,),
        in_specs=[pl.BlockSpec((_TC_STEP * 128, 128), lambda i: (i, 0))],
        out_specs=pl.BlockSpec(
            (8, 2, _TC_STEP, 8, 128),
            lambda i: (i // (_BT // _TC_STEP), 0, i % (_BT // _TC_STEP), 0, 0),
        ),
        out_shape=jax.ShapeDtypeStruct((_HIST, 2, _BT, 8, 128), jnp.float32),
    )(rows_view)


def kernel(pitch_type, table):
    idx_phys = (
        pitch_type.T.reshape(_HT, 8, _BT, 128)
        .transpose(0, 2, 1, 3)
        .reshape(_TILES, 1024)
        .astype(jnp.int32)
    )

    rows = _sc_gather(table, idx_phys)
    rows_view = rows.reshape(409600, 128)
    out_phys = _tc_permute(rows_view)

    return out_phys.transpose(2, 4, 0, 1, 3).reshape(_BATCH, _HIST, _EMBED)
